# Initial kernel scaffold; baseline (speedup 1.0000x reference)
import jax
import jax.numpy as jnp
from jax.experimental import pallas as pl


def kernel(x, edge_index, W1_l, b1, W1_r, W2_l, b2, W2_r):
    raise NotImplementedError("write your pallas kernel here")


# speedup vs baseline: 3.1696x; 3.1696x over previous
import functools

import jax
import jax.numpy as jnp
from jax import lax
from jax.experimental import pallas as pl
from jax.experimental.pallas import tpu as pltpu
from jax.experimental.pallas import tpu_sc as plsc

NC = 2
NS = 16
LB = 128


def _sc_mesh():
    return plsc.VectorSubcoreMesh(core_axis_name="c", subcore_axis_name="s")


def _make_agg(n, d, nblk_pad, with_gather):
    nw = NC * NS
    bpw = nblk_pad // nw
    grp = 8
    n_pad = ((n + LB - 1) // LB) * LB
    rps = n_pad // NS
    nfull = rps // LB
    tail = rps - nfull * LB

    scratch = [
        pltpu.VMEM((grp, LB), jnp.int32),
        pltpu.VMEM((grp, LB), jnp.int32),
        pltpu.VMEM((LB, d), jnp.float32),
        pltpu.VMEM_SHARED((n_pad, d), jnp.float32),
        pltpu.SemaphoreType.DMA,
    ]

    @functools.partial(
        pl.kernel, mesh=_sc_mesh(),
        out_type=jax.ShapeDtypeStruct((NC, n_pad, d), jnp.float32),
        scratch_types=scratch)
    def agg(x_hbm, src_hbm, dst_hbm, zrow_hbm, out_hbm,
            src_v, dst_v, rows_v, acc_sh, sem):
        cid = lax.axis_index("c")
        sid = lax.axis_index("s")
        wid = sid * NC + cid
        base_r = sid * rps

        pltpu.sync_copy(zrow_hbm, rows_v)
        for kk in range(nfull):
            pltpu.sync_copy(rows_v, acc_sh.at[pl.ds(base_r + kk * LB, LB)])
        if tail:
            pltpu.sync_copy(rows_v.at[pl.ds(0, tail)],
                            acc_sh.at[pl.ds(base_r + nfull * LB, tail)])
        if not with_gather:
            pltpu.sync_copy(x_hbm, rows_v)
        plsc.subcore_barrier()

        def group(kk, carry):
            pltpu.sync_copy(src_hbm.at[pl.ds(wid * bpw + kk * grp, grp)],
                            src_v)
            pltpu.sync_copy(dst_hbm.at[pl.ds(wid * bpw + kk * grp, grp)],
                            dst_v)

            def step(j, c):
                if with_gather:
                    pltpu.async_copy(x_hbm.at[src_v.at[j]], rows_v,
                                     sem).wait()
                pltpu.sync_copy(rows_v, acc_sh.at[dst_v.at[j]], add=True)
                return c

            return lax.fori_loop(0, grp, step, carry)

        lax.fori_loop(0, bpw // grp, group, 0)
        plsc.subcore_barrier()

        for kk in range(nfull):
            pltpu.sync_copy(acc_sh.at[pl.ds(base_r + kk * LB, LB)], rows_v)
            pltpu.sync_copy(rows_v,
                            out_hbm.at[cid, pl.ds(base_r + kk * LB, LB)])
        if tail:
            pltpu.sync_copy(acc_sh.at[pl.ds(base_r + nfull * LB, tail)],
                            rows_v.at[pl.ds(0, tail)])
            pltpu.sync_copy(rows_v.at[pl.ds(0, tail)],
                            out_hbm.at[cid, pl.ds(base_r + nfull * LB, tail)])

    return agg


def _dense_body(relu, p_ref, dp_ref, x_ref, wl_ref, wr_ref, b_ref, o_ref):
    s = p_ref[0] + p_ref[1]
    deg = dp_ref[0, :, 0:1] + dp_ref[1, :, 0:1]
    rdeg = 1.0 / jnp.maximum(deg, 1.0)
    acc = jnp.dot(s * rdeg, wl_ref[...], preferred_element_type=jnp.float32)
    acc = acc + jnp.dot(x_ref[...], wr_ref[...],
                        preferred_element_type=jnp.float32)
    acc = acc + b_ref[...]
    o_ref[...] = jnp.maximum(acc, 0.0) if relu else acc


def _make_dense(n, n_pad, d, relu, blk=1000):
    return pl.pallas_call(
        functools.partial(_dense_body, relu),
        grid=(n // blk,),
        in_specs=[
            pl.BlockSpec((NC, blk, d), lambda i: (0, i, 0)),
            pl.BlockSpec((NC, blk, d), lambda i: (0, i, 0)),
            pl.BlockSpec((blk, d), lambda i: (i, 0)),
            pl.BlockSpec((d, d), lambda i: (0, 0)),
            pl.BlockSpec((d, d), lambda i: (0, 0)),
            pl.BlockSpec((1, d), lambda i: (0, 0)),
        ],
        out_specs=pl.BlockSpec((blk, d), lambda i: (i, 0)),
        out_shape=jax.ShapeDtypeStruct((n, d), jnp.float32),
    )


def kernel(x, edge_index, W1_l, b1, W1_r, W2_l, b2, W2_r):
    n, d = x.shape
    e = edge_index.shape[1]
    nw = NC * NS
    n_pad = ((n + LB - 1) // LB) * LB
    nblk_pad = ((e + nw * 8 * LB - 1) // (nw * 8 * LB)) * nw * 8
    e_pad = nblk_pad * LB
    assert n % 8 == 0 and n % 1000 == 0

    pad = e_pad - e
    src = jnp.concatenate(
        [edge_index[0], jnp.zeros((pad,), jnp.int32)]).reshape(nblk_pad, LB)
    dst = jnp.concatenate(
        [edge_index[1], jnp.full((pad,), n, jnp.int32)]).reshape(nblk_pad, LB)
    ones_rows = jnp.ones((LB, d), jnp.float32)
    zrow = jnp.zeros((LB, d), jnp.float32)

    agg = _make_agg(n, d, nblk_pad, with_gather=True)
    deg_agg = _make_agg(n, d, nblk_pad, with_gather=False)
    dense1 = _make_dense(n, n_pad, d, relu=True)
    dense2 = _make_dense(n, n_pad, d, relu=False)

    degp = deg_agg(ones_rows, src, dst, zrow)
    s1 = agg(x, src, dst, zrow)
    h = dense1(s1, degp, x, W1_l.T, W1_r.T, b1.reshape(1, d))
    s2 = agg(h, src, dst, zrow)
    out = dense2(s2, degp, h, W2_l.T, W2_r.T, b2.reshape(1, d))
    return out


# speedup vs baseline: 3.4302x; 1.0822x over previous
import functools

import jax
import jax.numpy as jnp
from jax import lax
from jax.experimental import pallas as pl
from jax.experimental.pallas import tpu as pltpu
from jax.experimental.pallas import tpu_sc as plsc

NC = 2
NS = 16
LB = 128


def _sc_mesh():
    return plsc.VectorSubcoreMesh(core_axis_name="c", subcore_axis_name="s")


def _make_agg(n, d, nblk_pad, with_gather):
    nw = NC * NS
    bpw = nblk_pad // nw
    grp = 8
    n_pad = ((n + LB - 1) // LB) * LB
    rps = n_pad // NS
    nfull = rps // LB
    tail = rps - nfull * LB

    scratch = [
        pltpu.VMEM((grp, LB), jnp.int32),
        pltpu.VMEM((grp, LB), jnp.int32),
        pltpu.VMEM((LB, d), jnp.float32),
        pltpu.VMEM((LB, d), jnp.float32),
        pltpu.VMEM_SHARED((n_pad, d), jnp.float32),
        pltpu.SemaphoreType.DMA,
        pltpu.SemaphoreType.DMA,
    ]

    @functools.partial(
        pl.kernel, mesh=_sc_mesh(),
        out_type=jax.ShapeDtypeStruct((NC, n_pad, d), jnp.float32),
        scratch_types=scratch)
    def agg(x_hbm, src_hbm, dst_hbm, zrow_hbm, out_hbm,
            src_v, dst_v, rows_v, rows2_v, acc_sh, sem, sem2):
        cid = lax.axis_index("c")
        sid = lax.axis_index("s")
        wid = sid * NC + cid
        base_r = sid * rps

        pltpu.sync_copy(zrow_hbm, rows_v)
        for kk in range(nfull):
            pltpu.sync_copy(rows_v, acc_sh.at[pl.ds(base_r + kk * LB, LB)])
        if tail:
            pltpu.sync_copy(rows_v.at[pl.ds(0, tail)],
                            acc_sh.at[pl.ds(base_r + nfull * LB, tail)])
        if not with_gather:
            pltpu.sync_copy(x_hbm, rows_v)
        plsc.subcore_barrier()

        bufs = ((rows_v, sem), (rows2_v, sem2))

        def group(kk, carry):
            pltpu.sync_copy(src_hbm.at[pl.ds(wid * bpw + kk * grp, grp)],
                            src_v)
            pltpu.sync_copy(dst_hbm.at[pl.ds(wid * bpw + kk * grp, grp)],
                            dst_v)
            if with_gather:
                pend = pltpu.async_copy(x_hbm.at[src_v.at[0]], rows_v, sem)
                for j in range(grp):
                    buf = bufs[j % 2][0]
                    if j + 1 < grp:
                        nxt = pltpu.async_copy(
                            x_hbm.at[src_v.at[j + 1]],
                            bufs[(j + 1) % 2][0], bufs[(j + 1) % 2][1])
                    pend.wait()
                    pltpu.sync_copy(buf, acc_sh.at[dst_v.at[j]], add=True)
                    if j + 1 < grp:
                        pend = nxt
            else:
                def step(j, c):
                    pltpu.sync_copy(rows_v, acc_sh.at[dst_v.at[j]], add=True)
                    return c

                lax.fori_loop(0, grp, step, 0)
            return carry

        lax.fori_loop(0, bpw // grp, group, 0)
        plsc.subcore_barrier()

        for kk in range(nfull):
            pltpu.sync_copy(acc_sh.at[pl.ds(base_r + kk * LB, LB)], rows_v)
            pltpu.sync_copy(rows_v,
                            out_hbm.at[cid, pl.ds(base_r + kk * LB, LB)])
        if tail:
            pltpu.sync_copy(acc_sh.at[pl.ds(base_r + nfull * LB, tail)],
                            rows_v.at[pl.ds(0, tail)])
            pltpu.sync_copy(rows_v.at[pl.ds(0, tail)],
                            out_hbm.at[cid, pl.ds(base_r + nfull * LB, tail)])

    return agg


def _dense_body(relu, p_ref, dp_ref, x_ref, wl_ref, wr_ref, b_ref, o_ref):
    s = p_ref[0] + p_ref[1]
    deg = dp_ref[0, :, 0:1] + dp_ref[1, :, 0:1]
    rdeg = 1.0 / jnp.maximum(deg, 1.0)
    acc = jnp.dot(s * rdeg, wl_ref[...], preferred_element_type=jnp.float32)
    acc = acc + jnp.dot(x_ref[...], wr_ref[...],
                        preferred_element_type=jnp.float32)
    acc = acc + b_ref[...]
    o_ref[...] = jnp.maximum(acc, 0.0) if relu else acc


def _make_dense(n, n_pad, d, relu, blk=1000):
    return pl.pallas_call(
        functools.partial(_dense_body, relu),
        grid=(n // blk,),
        in_specs=[
            pl.BlockSpec((NC, blk, d), lambda i: (0, i, 0)),
            pl.BlockSpec((NC, blk, d), lambda i: (0, i, 0)),
            pl.BlockSpec((blk, d), lambda i: (i, 0)),
            pl.BlockSpec((d, d), lambda i: (0, 0)),
            pl.BlockSpec((d, d), lambda i: (0, 0)),
            pl.BlockSpec((1, d), lambda i: (0, 0)),
        ],
        out_specs=pl.BlockSpec((blk, d), lambda i: (i, 0)),
        out_shape=jax.ShapeDtypeStruct((n, d), jnp.float32),
    )


def kernel(x, edge_index, W1_l, b1, W1_r, W2_l, b2, W2_r):
    n, d = x.shape
    e = edge_index.shape[1]
    nw = NC * NS
    n_pad = ((n + LB - 1) // LB) * LB
    nblk_pad = ((e + nw * 8 * LB - 1) // (nw * 8 * LB)) * nw * 8
    e_pad = nblk_pad * LB
    assert n % 8 == 0 and n % 1000 == 0

    pad = e_pad - e
    src = jnp.concatenate(
        [edge_index[0], jnp.zeros((pad,), jnp.int32)]).reshape(nblk_pad, LB)
    dst = jnp.concatenate(
        [edge_index[1], jnp.full((pad,), n, jnp.int32)]).reshape(nblk_pad, LB)
    ones_rows = jnp.ones((LB, d), jnp.float32)
    zrow = jnp.zeros((LB, d), jnp.float32)

    agg = _make_agg(n, d, nblk_pad, with_gather=True)
    deg_agg = _make_agg(n, d, nblk_pad, with_gather=False)
    dense1 = _make_dense(n, n_pad, d, relu=True)
    dense2 = _make_dense(n, n_pad, d, relu=False)

    degp = deg_agg(ones_rows, src, dst, zrow)
    s1 = agg(x, src, dst, zrow)
    h = dense1(s1, degp, x, W1_l.T, W1_r.T, b1.reshape(1, d))
    s2 = agg(h, src, dst, zrow)
    out = dense2(s2, degp, h, W2_l.T, W2_r.T, b2.reshape(1, d))
    return out


# speedup vs baseline: 3.4495x; 1.0056x over previous
import functools

import jax
import jax.numpy as jnp
from jax import lax
from jax.experimental import pallas as pl
from jax.experimental.pallas import tpu as pltpu
from jax.experimental.pallas import tpu_sc as plsc

NC = 2
NS = 16
LB = 128


def _sc_mesh():
    return plsc.VectorSubcoreMesh(core_axis_name="c", subcore_axis_name="s")


def _make_agg(n, d, nblk_pad, with_gather):
    nw = NC * NS
    bpw = nblk_pad // nw
    grp = 16
    n_pad = ((n + LB - 1) // LB) * LB
    rps = n_pad // NS
    nfull = rps // LB
    tail = rps - nfull * LB

    hb = LB // 2

    scratch = [
        pltpu.VMEM((2 * grp, hb), jnp.int32),
        pltpu.VMEM((grp, LB), jnp.int32),
        pltpu.VMEM((LB, d), jnp.float32),
        pltpu.VMEM((LB, d), jnp.float32),
        pltpu.VMEM_SHARED((n_pad, d), jnp.float32),
        pltpu.SemaphoreType.DMA,
        pltpu.SemaphoreType.DMA,
        pltpu.SemaphoreType.DMA,
        pltpu.SemaphoreType.DMA,
    ]

    @functools.partial(
        pl.kernel, mesh=_sc_mesh(),
        out_type=jax.ShapeDtypeStruct((NC, n_pad, d), jnp.float32),
        scratch_types=scratch)
    def agg(x_hbm, src_hbm, dst_hbm, zrow_hbm, out_hbm,
            src_v, dst_v, rows_v, rows2_v, acc_sh,
            sem, sem2, sem3, sem4):
        cid = lax.axis_index("c")
        sid = lax.axis_index("s")
        wid = sid * NC + cid
        base_r = sid * rps

        pltpu.sync_copy(zrow_hbm, rows_v)
        for kk in range(nfull):
            pltpu.sync_copy(rows_v, acc_sh.at[pl.ds(base_r + kk * LB, LB)])
        if tail:
            pltpu.sync_copy(rows_v.at[pl.ds(0, tail)],
                            acc_sh.at[pl.ds(base_r + nfull * LB, tail)])
        if not with_gather:
            pltpu.sync_copy(x_hbm, rows_v)
        plsc.subcore_barrier()

        bufs = ((rows_v, sem, sem2), (rows2_v, sem3, sem4))

        def post(j):
            buf, s0, s1 = bufs[j % 2]
            p0 = pltpu.async_copy(x_hbm.at[src_v.at[2 * j]],
                                  buf.at[pl.ds(0, hb)], s0)
            p1 = pltpu.async_copy(x_hbm.at[src_v.at[2 * j + 1]],
                                  buf.at[pl.ds(hb, hb)], s1)
            return (p0, p1)

        def group(kk, carry):
            pltpu.sync_copy(
                src_hbm.at[pl.ds(2 * (wid * bpw + kk * grp), 2 * grp)],
                src_v)
            pltpu.sync_copy(dst_hbm.at[pl.ds(wid * bpw + kk * grp, grp)],
                            dst_v)
            if with_gather:
                pend = [None] * grp
                for j in range(min(2, grp)):
                    pend[j] = post(j)
                for j in range(grp):
                    pend[j][0].wait()
                    pend[j][1].wait()
                    pltpu.sync_copy(bufs[j % 2][0],
                                    acc_sh.at[dst_v.at[j]], add=True)
                    if j + 2 < grp:
                        pend[j + 2] = post(j + 2)
            else:
                def step(j, c):
                    pltpu.sync_copy(rows_v, acc_sh.at[dst_v.at[j]], add=True)
                    return c

                lax.fori_loop(0, grp, step, 0)
            return carry

        lax.fori_loop(0, bpw // grp, group, 0)
        plsc.subcore_barrier()

        for kk in range(nfull):
            pltpu.sync_copy(acc_sh.at[pl.ds(base_r + kk * LB, LB)], rows_v)
            pltpu.sync_copy(rows_v,
                            out_hbm.at[cid, pl.ds(base_r + kk * LB, LB)])
        if tail:
            pltpu.sync_copy(acc_sh.at[pl.ds(base_r + nfull * LB, tail)],
                            rows_v.at[pl.ds(0, tail)])
            pltpu.sync_copy(rows_v.at[pl.ds(0, tail)],
                            out_hbm.at[cid, pl.ds(base_r + nfull * LB, tail)])

    return agg


def _dense_body(relu, p_ref, dp_ref, x_ref, wl_ref, wr_ref, b_ref, o_ref):
    s = p_ref[0] + p_ref[1]
    deg = dp_ref[0, :, 0:1] + dp_ref[1, :, 0:1]
    rdeg = 1.0 / jnp.maximum(deg, 1.0)
    acc = jnp.dot(s * rdeg, wl_ref[...], preferred_element_type=jnp.float32)
    acc = acc + jnp.dot(x_ref[...], wr_ref[...],
                        preferred_element_type=jnp.float32)
    acc = acc + b_ref[...]
    o_ref[...] = jnp.maximum(acc, 0.0) if relu else acc


def _make_dense(n, n_pad, d, relu, blk=1000):
    return pl.pallas_call(
        functools.partial(_dense_body, relu),
        grid=(n // blk,),
        in_specs=[
            pl.BlockSpec((NC, blk, d), lambda i: (0, i, 0)),
            pl.BlockSpec((NC, blk, d), lambda i: (0, i, 0)),
            pl.BlockSpec((blk, d), lambda i: (i, 0)),
            pl.BlockSpec((d, d), lambda i: (0, 0)),
            pl.BlockSpec((d, d), lambda i: (0, 0)),
            pl.BlockSpec((1, d), lambda i: (0, 0)),
        ],
        out_specs=pl.BlockSpec((blk, d), lambda i: (i, 0)),
        out_shape=jax.ShapeDtypeStruct((n, d), jnp.float32),
    )


def kernel(x, edge_index, W1_l, b1, W1_r, W2_l, b2, W2_r):
    n, d = x.shape
    e = edge_index.shape[1]
    nw = NC * NS
    n_pad = ((n + LB - 1) // LB) * LB
    nblk_pad = ((e + nw * 16 * LB - 1) // (nw * 16 * LB)) * nw * 16
    e_pad = nblk_pad * LB
    assert n % 8 == 0 and n % 1000 == 0

    pad = e_pad - e
    src = jnp.concatenate(
        [edge_index[0],
         jnp.zeros((pad,), jnp.int32)]).reshape(2 * nblk_pad, LB // 2)
    dst = jnp.concatenate(
        [edge_index[1], jnp.full((pad,), n, jnp.int32)]).reshape(nblk_pad, LB)
    ones_rows = jnp.ones((LB, d), jnp.float32)
    zrow = jnp.zeros((LB, d), jnp.float32)

    agg = _make_agg(n, d, nblk_pad, with_gather=True)
    deg_agg = _make_agg(n, d, nblk_pad, with_gather=False)
    dense1 = _make_dense(n, n_pad, d, relu=True)
    dense2 = _make_dense(n, n_pad, d, relu=False)

    degp = deg_agg(ones_rows, src, dst, zrow)
    s1 = agg(x, src, dst, zrow)
    h = dense1(s1, degp, x, W1_l.T, W1_r.T, b1.reshape(1, d))
    s2 = agg(h, src, dst, zrow)
    out = dense2(s2, degp, h, W2_l.T, W2_r.T, b2.reshape(1, d))
    return out


# speedup vs baseline: 3.4679x; 1.0053x over previous
import functools

import jax
import jax.numpy as jnp
from jax import lax
from jax.experimental import pallas as pl
from jax.experimental.pallas import tpu as pltpu
from jax.experimental.pallas import tpu_sc as plsc

NC = 2
NS = 16
LB = 128


def _sc_mesh():
    return plsc.VectorSubcoreMesh(core_axis_name="c", subcore_axis_name="s")


def _make_agg(n, d, nblk_pad, with_gather):
    nw = NC * NS
    bpw = nblk_pad // nw
    grp = 16
    n_pad = ((n + LB - 1) // LB) * LB
    rps = n_pad // NS

    hb = LB // 2

    scratch = [
        pltpu.VMEM((2 * grp, hb), jnp.int32),
        pltpu.VMEM((grp, LB), jnp.int32),
        pltpu.VMEM((LB, d), jnp.float32),
        pltpu.VMEM((LB, d), jnp.float32),
        pltpu.VMEM_SHARED((n_pad, d), jnp.float32),
        pltpu.SemaphoreType.DMA,
        pltpu.SemaphoreType.DMA,
        pltpu.SemaphoreType.DMA,
        pltpu.SemaphoreType.DMA,
    ]

    @functools.partial(
        pl.kernel, mesh=_sc_mesh(),
        out_type=jax.ShapeDtypeStruct((NC, n_pad, d), jnp.float32),
        scratch_types=scratch)
    def agg(x_hbm, src_hbm, dst_hbm, zrow_hbm, out_hbm,
            src_v, dst_v, rows_v, rows2_v, acc_sh,
            sem, sem2, sem3, sem4):
        cid = lax.axis_index("c")
        sid = lax.axis_index("s")
        wid = sid * NC + cid
        base_r = sid * rps

        pltpu.sync_copy(zrow_hbm, acc_sh.at[pl.ds(base_r, rps)])
        if not with_gather:
            pltpu.sync_copy(x_hbm, rows_v)
        plsc.subcore_barrier()

        bufs = ((rows_v, sem, sem2), (rows2_v, sem3, sem4))

        def post(j):
            buf, s0, s1 = bufs[j % 2]
            p0 = pltpu.async_copy(x_hbm.at[src_v.at[2 * j]],
                                  buf.at[pl.ds(0, hb)], s0)
            p1 = pltpu.async_copy(x_hbm.at[src_v.at[2 * j + 1]],
                                  buf.at[pl.ds(hb, hb)], s1)
            return (p0, p1)

        def group(kk, carry):
            pltpu.sync_copy(
                src_hbm.at[pl.ds(2 * (wid * bpw + kk * grp), 2 * grp)],
                src_v)
            pltpu.sync_copy(dst_hbm.at[pl.ds(wid * bpw + kk * grp, grp)],
                            dst_v)
            if with_gather:
                pend = [None] * grp
                for j in range(min(2, grp)):
                    pend[j] = post(j)
                for j in range(grp):
                    pend[j][0].wait()
                    pend[j][1].wait()
                    pltpu.sync_copy(bufs[j % 2][0],
                                    acc_sh.at[dst_v.at[j]], add=True)
                    if j + 2 < grp:
                        pend[j + 2] = post(j + 2)
            else:
                def step(j, c):
                    pltpu.sync_copy(rows_v, acc_sh.at[dst_v.at[j]], add=True)
                    return c

                lax.fori_loop(0, grp, step, 0)
            return carry

        lax.fori_loop(0, bpw // grp, group, 0)
        plsc.subcore_barrier()

        pltpu.sync_copy(acc_sh.at[pl.ds(base_r, rps)],
                        out_hbm.at[cid, pl.ds(base_r, rps)])

    return agg


def _dense_body(relu, p_ref, dp_ref, x_ref, wl_ref, wr_ref, b_ref, o_ref):
    s = p_ref[0] + p_ref[1]
    deg = dp_ref[0, :, 0:1] + dp_ref[1, :, 0:1]
    rdeg = 1.0 / jnp.maximum(deg, 1.0)
    acc = jnp.dot(s * rdeg, wl_ref[...], preferred_element_type=jnp.float32)
    acc = acc + jnp.dot(x_ref[...], wr_ref[...],
                        preferred_element_type=jnp.float32)
    acc = acc + b_ref[...]
    o_ref[...] = jnp.maximum(acc, 0.0) if relu else acc


def _make_dense(n, n_pad, d, relu, blk=1000):
    return pl.pallas_call(
        functools.partial(_dense_body, relu),
        grid=(n // blk,),
        in_specs=[
            pl.BlockSpec((NC, blk, d), lambda i: (0, i, 0)),
            pl.BlockSpec((NC, blk, d), lambda i: (0, i, 0)),
            pl.BlockSpec((blk, d), lambda i: (i, 0)),
            pl.BlockSpec((d, d), lambda i: (0, 0)),
            pl.BlockSpec((d, d), lambda i: (0, 0)),
            pl.BlockSpec((1, d), lambda i: (0, 0)),
        ],
        out_specs=pl.BlockSpec((blk, d), lambda i: (i, 0)),
        out_shape=jax.ShapeDtypeStruct((n, d), jnp.float32),
    )


def kernel(x, edge_index, W1_l, b1, W1_r, W2_l, b2, W2_r):
    n, d = x.shape
    e = edge_index.shape[1]
    nw = NC * NS
    n_pad = ((n + LB - 1) // LB) * LB
    nblk_pad = ((e + nw * 16 * LB - 1) // (nw * 16 * LB)) * nw * 16
    e_pad = nblk_pad * LB
    assert n % 8 == 0 and n % 1000 == 0

    pad = e_pad - e
    src = jnp.concatenate(
        [edge_index[0],
         jnp.zeros((pad,), jnp.int32)]).reshape(2 * nblk_pad, LB // 2)
    dst = jnp.concatenate(
        [edge_index[1], jnp.full((pad,), n, jnp.int32)]).reshape(nblk_pad, LB)
    ones_rows = jnp.ones((LB, d), jnp.float32)
    zrow = jnp.zeros((n_pad // NS, d), jnp.float32)

    agg = _make_agg(n, d, nblk_pad, with_gather=True)
    deg_agg = _make_agg(n, d, nblk_pad, with_gather=False)
    dense1 = _make_dense(n, n_pad, d, relu=True)
    dense2 = _make_dense(n, n_pad, d, relu=False)

    degp = deg_agg(ones_rows, src, dst, zrow)
    s1 = agg(x, src, dst, zrow)
    h = dense1(s1, degp, x, W1_l.T, W1_r.T, b1.reshape(1, d))
    s2 = agg(h, src, dst, zrow)
    out = dense2(s2, degp, h, W2_l.T, W2_r.T, b2.reshape(1, d))
    return out
